# baseline (speedup 1.0000x reference)
import jax
import jax.numpy as jnp
from jax import lax
from jax.experimental import pallas as pl
from jax.experimental.pallas import tpu as pltpu, tpu_sc as plsc

NC, NS, L = 2, 16, 16
NW = NC * NS
D = 768
SEQ = 2048
B = 4
R = B * SEQ
RPW = R // NW
PPW = SEQ // NW
K = 16
NCHUNK = RPW // K
NPAIR = NCHUNK // 2
NQ = 4
NG = D // L
EPS = 1e-12
INV_D = 1.0 / D


def _rsqrt_vec(x):
    i = lax.bitcast_convert_type(x, jnp.int32)
    i = jnp.int32(0x5F3759DF) - (i >> 1)
    y = lax.bitcast_convert_type(i, jnp.float32)
    for _ in range(3):
        y = y * (1.5 - 0.5 * x * y * y)
    return y


def _lane_sum2(a, b):
    lanes = lax.iota(jnp.int32, L)
    for k in (1, 2, 4, 8):
        idx = lanes ^ k
        a = a + a.at[idx].get(mode="promise_in_bounds")
        b = b + b.at[idx].get(mode="promise_in_bounds")
    return a, b


def _stats_tail(fin):
    mus, rss = [], []
    for slot in range(K):
        s, s2 = _lane_sum2(fin[slot], fin[K + slot])
        mu = s * INV_D
        mus.append(mu)
        rss.append(_rsqrt_vec(s2 * INV_D - mu * mu + EPS))
    return mus, rss


def _chunk_compute(c, buf, pos_v, gamma_v, beta_v, obuf):
    c4d = (c * NQ) * D

    def stat_body(j, carry):
        jl = j * L
        ps = [pos_v[pl.ds(c4d + q * D + jl, L)] for q in range(NQ)]
        new_s, new_s2 = [], []
        for q in range(NQ):
            for b in range(B):
                slot = NQ * q + b
                v = buf[slot, pl.ds(jl, L)] + ps[q]
                new_s.append(carry[slot] + v)
                new_s2.append(carry[K + slot] + v * v)
        return tuple(new_s + new_s2)

    zeros = tuple(jnp.zeros((L,), jnp.float32) for _ in range(2 * K))
    fin = lax.fori_loop(0, NG, stat_body, zeros)
    mus, rss = _stats_tail(fin)

    def norm_body(j, _):
        jl = j * L
        g = gamma_v[pl.ds(jl, L)]
        be = beta_v[pl.ds(jl, L)]
        ps = [pos_v[pl.ds(c4d + q * D + jl, L)] for q in range(NQ)]
        for q in range(NQ):
            for b in range(B):
                slot = NQ * q + b
                v = buf[slot, pl.ds(jl, L)] + ps[q]
                obuf[pl.ds((NQ * b + q) * D + jl, L)] = (
                    (v - mus[slot]) * rss[slot] * g + be)
        return 0

    lax.fori_loop(0, NG, norm_body, 0)


def _sc_body(ids_hbm, word_hbm, pos_hbm, gamma_hbm, beta_hbm, out_hbm,
             idx_v, pos_v, buf0, buf1, obuf0, obuf1,
             gamma_v, beta_v,
             gsem0, gsem1, osem0, osem1):
    wid = lax.axis_index("s") * NC + lax.axis_index("c")
    s_base = wid * PPW
    pltpu.sync_copy(pos_hbm.at[pl.ds(s_base * D, PPW * D)], pos_v)
    pltpu.sync_copy(ids_hbm.at[pl.ds(wid * RPW, RPW)], idx_v)
    pltpu.sync_copy(gamma_hbm, gamma_v)
    pltpu.sync_copy(beta_hbm, beta_v)

    def issue_g(c, bufs, gsem):
        pltpu.async_copy(word_hbm.at[idx_v.at[pl.ds(c * K, K)]], bufs, gsem)

    def wait_g(c, bufs, gsem):
        pltpu.make_async_copy(word_hbm.at[idx_v.at[pl.ds(c * K, K)]], bufs,
                              gsem).wait()

    def start_out(c, obufs, osem):
        for b in range(B):
            pltpu.async_copy(
                obufs.at[pl.ds(b * (NQ * D), NQ * D)],
                out_hbm.at[pl.ds((b * SEQ + s_base + NQ * c) * D, NQ * D)],
                osem)

    def wait_out(c, obufs, osem):
        for b in range(B):
            pltpu.make_async_copy(
                obufs.at[pl.ds(b * (NQ * D), NQ * D)],
                out_hbm.at[pl.ds((b * SEQ + s_base + NQ * c) * D, NQ * D)],
                osem).wait()

    issue_g(0, buf0, gsem0)

    def pair(t, _):
        c0 = 2 * t
        issue_g(c0 + 1, buf1, gsem1)
        wait_g(c0, buf0, gsem0)

        @pl.when(t > 0)
        def _():
            wait_out(c0 - 2, obuf0, osem0)

        _chunk_compute(c0, buf0, pos_v, gamma_v, beta_v, obuf0)
        start_out(c0, obuf0, osem0)

        @pl.when(t < NPAIR - 1)
        def _():
            issue_g(c0 + 2, buf0, gsem0)
        wait_g(c0 + 1, buf1, gsem1)

        @pl.when(t > 0)
        def _():
            wait_out(c0 - 1, obuf1, osem1)

        _chunk_compute(c0 + 1, buf1, pos_v, gamma_v, beta_v, obuf1)
        start_out(c0 + 1, obuf1, osem1)
        return 0

    lax.fori_loop(0, NPAIR, pair, 0)

    c_last = 2 * (NPAIR - 1)
    wait_out(c_last, obuf0, osem0)
    wait_out(c_last + 1, obuf1, osem1)


_sc_embed = pl.kernel(
    _sc_body,
    out_type=jax.ShapeDtypeStruct((R * D,), jnp.float32),
    mesh=plsc.VectorSubcoreMesh(core_axis_name="c", subcore_axis_name="s"),
    scratch_types=[
        pltpu.VMEM((RPW,), jnp.int32),
        pltpu.VMEM((PPW * D,), jnp.float32),
        pltpu.VMEM((K, D), jnp.float32),
        pltpu.VMEM((K, D), jnp.float32),
        pltpu.VMEM((K * D,), jnp.float32),
        pltpu.VMEM((K * D,), jnp.float32),
        pltpu.VMEM((D,), jnp.float32),
        pltpu.VMEM((D,), jnp.float32),
        pltpu.SemaphoreType.DMA,
        pltpu.SemaphoreType.DMA,
        pltpu.SemaphoreType.DMA,
        pltpu.SemaphoreType.DMA,
    ],
)


@jax.jit
def kernel(input_ids, word_emb, pos_emb, ln_weight, ln_bias):
    ids = input_ids.reshape(B, NW, NCHUNK, NQ).transpose(1, 2, 3, 0)
    out = _sc_embed(ids.reshape(-1), word_emb, pos_emb.reshape(-1),
                    ln_weight, ln_bias)
    return out.reshape(input_ids.shape + (D,))
